# baseline (speedup 1.0000x reference)
import jax
import jax.numpy as jnp
from jax.experimental import pallas as pl
from jax.experimental.pallas import tpu as pltpu

_BM = 200
_NBUF = 4


def _gcn_kernel(x_ref, adj_hbm, w_ref, b_ref, out_ref, xw_ref, buf, sems):
    n = adj_hbm.shape[0]
    nchunks = n // _BM

    def _copy(chunk, slot):
        return pltpu.make_async_copy(
            adj_hbm.at[pl.ds(chunk * _BM, _BM), :], buf.at[slot], sems.at[slot]
        )

    for j in range(min(_NBUF, nchunks)):
        _copy(j, j).start()

    xw_ref[...] = jax.lax.dot(
        x_ref[...], w_ref[...], preferred_element_type=jnp.float32
    )

    def step(i, carry):
        slot = jax.lax.rem(i, _NBUF)
        _copy(i, slot).wait()
        acc = jax.lax.dot(buf[slot], xw_ref[...],
                          preferred_element_type=jnp.float32)
        out_ref[pl.ds(i * _BM, _BM), :] = acc + b_ref[...]

        @pl.when(i + _NBUF < nchunks)
        def _prefetch():
            _copy(i + _NBUF, slot).start()

        return carry

    jax.lax.fori_loop(0, nchunks, step, 0)


def kernel(input, adj, weight, bias):
    n, f_in = input.shape
    f_out = weight.shape[1]
    bias2 = bias.reshape(1, f_out)
    return pl.pallas_call(
        _gcn_kernel,
        in_specs=[
            pl.BlockSpec(memory_space=pltpu.VMEM),
            pl.BlockSpec(memory_space=pl.ANY),
            pl.BlockSpec(memory_space=pltpu.VMEM),
            pl.BlockSpec(memory_space=pltpu.VMEM),
        ],
        out_specs=pl.BlockSpec(memory_space=pltpu.VMEM),
        out_shape=jax.ShapeDtypeStruct((n, f_out), jnp.float32),
        scratch_shapes=[
            pltpu.VMEM((n, f_out), jnp.float32),
            pltpu.VMEM((_NBUF, _BM, n), jnp.float32),
            pltpu.SemaphoreType.DMA((_NBUF,)),
        ],
    )(input, adj, weight, bias2)
